# baseline (speedup 1.0000x reference)
import jax
import jax.numpy as jnp
from jax import lax
from jax.experimental import pallas as pl
from jax.experimental.pallas import tpu as pltpu
from jax.experimental.pallas import tpu_sc as plsc

N_CHARS = 1000
EMB_DIM = 32
BATCH = 16384
SEQ = 200

_INFO = plsc.get_sparse_core_info()
_NC = _INFO.num_cores
_NS = _INFO.num_subcores
_NW = _NC * _NS

_CROWS = 8
_ROWS_W = BATCH // _NW
_STEPS = _ROWS_W // _CROWS


def _gather_kernel(sent_hbm, table_hbm, out3_hbm,
                   idx0, idx1, rows0, rows1,
                   sem_i0, sem_i1, sem_g0, sem_g1, sem_o0, sem_o1):
    wid = lax.axis_index("s") * _NC + lax.axis_index("c")
    rbase = wid * _ROWS_W

    def idx_load(g, buf, sem):
        pltpu.async_copy(
            sent_hbm.at[pl.ds(rbase + g * _CROWS, _CROWS)], buf, sem)

    def idx_wait(buf, sem):
        pltpu.make_async_copy(
            sent_hbm.at[pl.ds(rbase, _CROWS)], buf, sem).wait()

    def gather(buf_idx, buf_rows, sem):
        for r in range(_CROWS):
            pltpu.async_copy(
                table_hbm.at[buf_idx.at[r]], buf_rows.at[r], sem)

    def gather_wait(buf_idx, buf_rows, sem):
        for r in range(_CROWS):
            pltpu.make_async_copy(
                table_hbm.at[buf_idx.at[r]], buf_rows.at[r], sem).wait()

    def out_write(g, buf, sem):
        pltpu.async_copy(
            buf, out3_hbm.at[pl.ds(rbase + g * _CROWS, _CROWS)], sem)

    def out_wait(buf, sem):
        pltpu.make_async_copy(
            buf, out3_hbm.at[pl.ds(rbase, _CROWS)], sem).wait()

    idx_load(0, idx0, sem_i0)
    idx_load(1, idx1, sem_i1)
    out_write(1, rows1, sem_o1)
    idx_wait(idx0, sem_i0)
    gather(idx0, rows0, sem_g0)

    def body(p, carry):
        g0 = 2 * p
        gather_wait(idx0, rows0, sem_g0)
        idx_load(g0 + 2, idx0, sem_i0)
        out_wait(rows1, sem_o1)
        idx_wait(idx1, sem_i1)
        gather(idx1, rows1, sem_g1)
        out_write(g0, rows0, sem_o0)
        gather_wait(idx1, rows1, sem_g1)
        idx_load(g0 + 3, idx1, sem_i1)
        out_wait(rows0, sem_o0)
        idx_wait(idx0, sem_i0)
        gather(idx0, rows0, sem_g0)
        out_write(g0 + 1, rows1, sem_o1)
        return carry

    lax.fori_loop(0, (_STEPS - 2) // 2, body, 0, unroll=False)

    g_last = _STEPS - 2
    gather_wait(idx0, rows0, sem_g0)
    out_wait(rows1, sem_o1)
    idx_wait(idx1, sem_i1)
    gather(idx1, rows1, sem_g1)
    out_write(g_last, rows0, sem_o0)
    gather_wait(idx1, rows1, sem_g1)
    out_wait(rows0, sem_o0)
    out_write(g_last + 1, rows1, sem_o1)
    out_wait(rows1, sem_o1)


@jax.jit
def kernel(sentence, table):
    mesh = plsc.VectorSubcoreMesh(core_axis_name="c", subcore_axis_name="s")
    return pl.kernel(
        _gather_kernel,
        out_type=jax.ShapeDtypeStruct((BATCH, SEQ, EMB_DIM), jnp.float32),
        mesh=mesh,
        scratch_types=[
            pltpu.VMEM((_CROWS, SEQ), jnp.int32),
            pltpu.VMEM((_CROWS, SEQ), jnp.int32),
            pltpu.VMEM((_CROWS, SEQ, EMB_DIM), jnp.float32),
            pltpu.VMEM((_CROWS, SEQ, EMB_DIM), jnp.float32),
            pltpu.SemaphoreType.DMA,
            pltpu.SemaphoreType.DMA,
            pltpu.SemaphoreType.DMA,
            pltpu.SemaphoreType.DMA,
            pltpu.SemaphoreType.DMA,
            pltpu.SemaphoreType.DMA,
        ],
        compiler_params=pltpu.CompilerParams(use_tc_tiling_on_sc=False),
    )(sentence, table)


# speedup vs baseline: 2.7168x; 2.7168x over previous
import jax
import jax.numpy as jnp
from jax import lax
from jax.experimental import pallas as pl
from jax.experimental.pallas import tpu as pltpu
from jax.experimental.pallas import tpu_sc as plsc

N_CHARS = 1000
EMB_DIM = 32
BATCH = 16384
SEQ = 200

_INFO = plsc.get_sparse_core_info()
_NC = _INFO.num_cores
_NS = _INFO.num_subcores
_NW = _NC * _NS
_L = _INFO.num_lanes

_ST = SEQ // 8
_BT = BATCH // 128
_BT_W = _BT // _NW
_UNITS = _ST * _BT_W


def _emb_kernel(sent_hbm, table_hbm, out_hbm,
                table_v, idx0, idx1, obuf0, obuf1,
                sem_i0, sem_i1, sem_o0, sem_o1):
    wid = lax.axis_index("s") * _NC + lax.axis_index("c")

    def unit_coords(u):
        st = u // _BT_W
        bt = wid * _BT_W + u % _BT_W
        return st, bt

    def idx_load(u, buf, sem):
        st, bt = unit_coords(u)
        pltpu.async_copy(
            sent_hbm.at[pl.ds(st * 8, 8), pl.ds(bt * 128, 128)], buf, sem)

    def idx_wait(buf, sem):
        pltpu.make_async_copy(
            sent_hbm.at[pl.ds(0, 8), pl.ds(0, 128)], buf, sem).wait()

    def out_write(u, buf, sem):
        st, bt = unit_coords(u)
        pltpu.async_copy(
            buf, out_hbm.at[pl.ds(st * 8, 8), :, pl.ds(bt * 128, 128)], sem)

    def out_wait(buf, sem):
        pltpu.make_async_copy(
            buf, out_hbm.at[pl.ds(0, 8), :, pl.ds(0, 128)], sem).wait()

    def compute(ibuf, obuf):
        def grp(g, carry):
            s = g >> 3
            b0 = (g & 7) * _L
            v_idx = ibuf[s, pl.ds(b0, _L)]
            for e in range(EMB_DIM):
                e_vec = jnp.full((_L,), e, jnp.int32)
                col = plsc.load_gather(table_v, [e_vec, v_idx])
                obuf[s, e, pl.ds(b0, _L)] = col
            return carry

        lax.fori_loop(0, 64, grp, 0, unroll=False)

    pltpu.sync_copy(table_hbm, table_v)

    idx_load(0, idx0, sem_i0)
    idx_load(1, idx1, sem_i1)
    out_write(0, obuf0, sem_o0)
    out_write(1, obuf1, sem_o1)

    def body(p, carry):
        u0 = 2 * p
        out_wait(obuf0, sem_o0)
        idx_wait(idx0, sem_i0)
        compute(idx0, obuf0)
        idx_load(u0 + 2, idx0, sem_i0)
        out_write(u0, obuf0, sem_o0)
        out_wait(obuf1, sem_o1)
        idx_wait(idx1, sem_i1)
        compute(idx1, obuf1)
        idx_load(u0 + 3, idx1, sem_i1)
        out_write(u0 + 1, obuf1, sem_o1)
        return carry

    lax.fori_loop(0, (_UNITS - 2) // 2, body, 0, unroll=False)

    out_wait(obuf0, sem_o0)
    idx_wait(idx0, sem_i0)
    compute(idx0, obuf0)
    out_write(_UNITS - 2, obuf0, sem_o0)
    out_wait(obuf1, sem_o1)
    idx_wait(idx1, sem_i1)
    compute(idx1, obuf1)
    out_write(_UNITS - 1, obuf1, sem_o1)
    out_wait(obuf0, sem_o0)
    out_wait(obuf1, sem_o1)


@jax.jit
def kernel(sentence, table):
    sent_t = sentence.T
    table_t = table.T
    mesh = plsc.VectorSubcoreMesh(core_axis_name="c", subcore_axis_name="s")
    tmp = pl.kernel(
        _emb_kernel,
        out_type=jax.ShapeDtypeStruct((SEQ, EMB_DIM, BATCH), jnp.float32),
        mesh=mesh,
        scratch_types=[
            pltpu.VMEM((EMB_DIM, N_CHARS), jnp.float32),
            pltpu.VMEM((8, 128), jnp.int32),
            pltpu.VMEM((8, 128), jnp.int32),
            pltpu.VMEM((8, EMB_DIM, 128), jnp.float32),
            pltpu.VMEM((8, EMB_DIM, 128), jnp.float32),
            pltpu.SemaphoreType.DMA,
            pltpu.SemaphoreType.DMA,
            pltpu.SemaphoreType.DMA,
            pltpu.SemaphoreType.DMA,
        ],
        compiler_params=pltpu.CompilerParams(
            use_tc_tiling_on_sc=True, needs_layout_passes=False),
    )(sent_t, table_t)
    return jnp.transpose(tmp, (2, 0, 1))


# speedup vs baseline: 2.7460x; 1.0107x over previous
import jax
import jax.numpy as jnp
from jax import lax
from jax.experimental import pallas as pl
from jax.experimental.pallas import tpu as pltpu
from jax.experimental.pallas import tpu_sc as plsc

N_CHARS = 1000
EMB_DIM = 32
BATCH = 16384
SEQ = 200

_INFO = plsc.get_sparse_core_info()
_NC = _INFO.num_cores
_NS = _INFO.num_subcores
_NW = _NC * _NS
_L = _INFO.num_lanes

_SROWS = 4
_ST = SEQ // _SROWS
_BT = BATCH // 128
_BT_W = _BT // _NW
_UNITS = _ST * _BT_W
_FSTRIDE = 1008


def _emb_kernel(sent_hbm, table_hbm, out_hbm,
                table_v, table_f, idx0, idx1, obuf0, obuf1,
                sem_i0, sem_i1, sem_o0, sem_o1):
    wid = lax.axis_index("s") * _NC + lax.axis_index("c")

    def unit_coords(u):
        st = u // _BT_W
        bt = wid * _BT_W + u % _BT_W
        return st, bt

    def idx_load(u, buf, sem):
        st, bt = unit_coords(u)
        pltpu.async_copy(
            sent_hbm.at[pl.ds(st * _SROWS, _SROWS), pl.ds(bt * 128, 128)],
            buf, sem)

    def idx_wait(buf, sem):
        pltpu.make_async_copy(
            sent_hbm.at[pl.ds(0, _SROWS), pl.ds(0, 128)], buf, sem).wait()

    def out_write(u, buf, sem):
        st, bt = unit_coords(u)
        pltpu.async_copy(
            buf,
            out_hbm.at[pl.ds(st * _SROWS, _SROWS), :, pl.ds(bt * 128, 128)],
            sem)

    def out_wait(buf, sem):
        pltpu.make_async_copy(
            buf, out_hbm.at[pl.ds(0, _SROWS), :, pl.ds(0, 128)], sem).wait()

    def compute(ibuf, obuf):
        def srow(s, carry):
            vs = [ibuf[s, pl.ds(bs * _L, _L)] for bs in range(8)]
            for e in range(EMB_DIM):
                off = e * _FSTRIDE
                for bs in range(8):
                    col = plsc.load_gather(table_f, [vs[bs] + off])
                    obuf[s, e, pl.ds(bs * _L, _L)] = col
            return carry

        lax.fori_loop(0, _SROWS, srow, 0, unroll=False)

    pltpu.sync_copy(table_hbm, table_v)
    iota = lax.iota(jnp.int32, _L)

    def repack_row(e, carry):
        e_vec = jnp.full((_L,), 0, jnp.int32) + e
        for c in range(_FSTRIDE // _L):
            col = plsc.load_gather(table_v, [e_vec, iota + (c * _L)])
            table_f[pl.ds(e * _FSTRIDE + c * _L, _L)] = col
        return carry

    lax.fori_loop(0, EMB_DIM, repack_row, 0, unroll=False)

    idx_load(0, idx0, sem_i0)
    idx_load(1, idx1, sem_i1)
    out_write(0, obuf0, sem_o0)
    out_write(1, obuf1, sem_o1)

    def body(p, carry):
        u0 = 2 * p
        out_wait(obuf0, sem_o0)
        idx_wait(idx0, sem_i0)
        compute(idx0, obuf0)
        idx_load(u0 + 2, idx0, sem_i0)
        out_write(u0, obuf0, sem_o0)
        out_wait(obuf1, sem_o1)
        idx_wait(idx1, sem_i1)
        compute(idx1, obuf1)
        idx_load(u0 + 3, idx1, sem_i1)
        out_write(u0 + 1, obuf1, sem_o1)
        return carry

    lax.fori_loop(0, (_UNITS - 2) // 2, body, 0, unroll=False)

    out_wait(obuf0, sem_o0)
    idx_wait(idx0, sem_i0)
    compute(idx0, obuf0)
    out_write(_UNITS - 2, obuf0, sem_o0)
    out_wait(obuf1, sem_o1)
    idx_wait(idx1, sem_i1)
    compute(idx1, obuf1)
    out_write(_UNITS - 1, obuf1, sem_o1)
    out_wait(obuf0, sem_o0)
    out_wait(obuf1, sem_o1)


@jax.jit
def kernel(sentence, table):
    sent_t = sentence.T
    table_t = table.T
    mesh = plsc.VectorSubcoreMesh(core_axis_name="c", subcore_axis_name="s")
    tmp = pl.kernel(
        _emb_kernel,
        out_type=jax.ShapeDtypeStruct((SEQ, EMB_DIM, BATCH), jnp.float32),
        mesh=mesh,
        scratch_types=[
            pltpu.VMEM((EMB_DIM, N_CHARS), jnp.float32),
            pltpu.VMEM((EMB_DIM * _FSTRIDE,), jnp.float32),
            pltpu.VMEM((_SROWS, 128), jnp.int32),
            pltpu.VMEM((_SROWS, 128), jnp.int32),
            pltpu.VMEM((_SROWS, EMB_DIM, 128), jnp.float32),
            pltpu.VMEM((_SROWS, EMB_DIM, 128), jnp.float32),
            pltpu.SemaphoreType.DMA,
            pltpu.SemaphoreType.DMA,
            pltpu.SemaphoreType.DMA,
            pltpu.SemaphoreType.DMA,
        ],
        compiler_params=pltpu.CompilerParams(
            use_tc_tiling_on_sc=True, needs_layout_passes=False),
    )(sent_t, table_t)
    return jnp.transpose(tmp, (2, 0, 1))


# speedup vs baseline: 8.6154x; 3.1374x over previous
import jax
import jax.numpy as jnp
from jax import lax
from jax.experimental import pallas as pl
from jax.experimental.pallas import tpu as pltpu
from jax.experimental.pallas import tpu_sc as plsc

N_CHARS = 1000
EMB_DIM = 32
BATCH = 16384
SEQ = 200

_INFO = plsc.get_sparse_core_info()
_NC = _INFO.num_cores
_NS = _INFO.num_subcores
_NW = _NC * _NS
_L = _INFO.num_lanes

_SROWS = 4
_ST = SEQ // _SROWS
_BT = BATCH // 128
_BT_W = _BT // _NW
_UNITS = _ST * _BT_W
_FSTRIDE = 1008


def _emb_kernel(sent_hbm, table_hbm, out_hbm,
                table_v, table_f, idx0, idx1, obuf0, obuf1,
                sem_i0, sem_i1, sem_o0, sem_o1):
    wid = lax.axis_index("s") * _NC + lax.axis_index("c")

    def unit_coords(u):
        st = u // _BT_W
        bt = wid * _BT_W + u % _BT_W
        return st, bt

    def idx_load(u, buf, sem):
        st, bt = unit_coords(u)
        pltpu.async_copy(
            sent_hbm.at[pl.ds(st * _SROWS, _SROWS), pl.ds(bt * 128, 128)],
            buf, sem)

    def idx_wait(buf, sem):
        pltpu.make_async_copy(
            sent_hbm.at[pl.ds(0, _SROWS), pl.ds(0, 128)], buf, sem).wait()

    def out_write(u, buf, sem):
        st, bt = unit_coords(u)
        pltpu.async_copy(
            buf,
            out_hbm.at[pl.ds(st * _SROWS, _SROWS), :, pl.ds(bt * 128, 128)],
            sem)

    def out_wait(buf, sem):
        pltpu.make_async_copy(
            buf, out_hbm.at[pl.ds(0, _SROWS), :, pl.ds(0, 128)], sem).wait()

    def compute(ibuf, obuf):
        def srow(s, carry):
            vs = [ibuf[s, pl.ds(bs * _L, _L)] for bs in range(8)]
            for e in range(EMB_DIM):
                off = e * _FSTRIDE
                cols = [plsc.load_gather(table_f, [vs[bs] + off])
                        for bs in range(8)]
                for bs in range(8):
                    obuf[s, e, pl.ds(bs * _L, _L)] = cols[bs]
            return carry

        lax.fori_loop(0, _SROWS, srow, 0, unroll=False)

    pltpu.sync_copy(table_hbm, table_v)
    iota = lax.iota(jnp.int32, _L)

    def repack_row(e, carry):
        e_vec = jnp.full((_L,), 0, jnp.int32) + e
        for c in range(_FSTRIDE // _L):
            col = plsc.load_gather(table_v, [e_vec, iota + (c * _L)])
            table_f[pl.ds(e * _FSTRIDE + c * _L, _L)] = col
        return carry

    lax.fori_loop(0, EMB_DIM, repack_row, 0, unroll=False)

    idx_load(0, idx0, sem_i0)
    idx_load(1, idx1, sem_i1)
    out_write(0, obuf0, sem_o0)
    out_write(1, obuf1, sem_o1)

    def body(p, carry):
        u0 = 2 * p
        out_wait(obuf0, sem_o0)
        idx_wait(idx0, sem_i0)
        compute(idx0, obuf0)
        idx_load(u0 + 2, idx0, sem_i0)
        out_write(u0, obuf0, sem_o0)
        out_wait(obuf1, sem_o1)
        idx_wait(idx1, sem_i1)
        compute(idx1, obuf1)
        idx_load(u0 + 3, idx1, sem_i1)
        out_write(u0 + 1, obuf1, sem_o1)
        return carry

    lax.fori_loop(0, (_UNITS - 2) // 2, body, 0, unroll=False)

    out_wait(obuf0, sem_o0)
    idx_wait(idx0, sem_i0)
    compute(idx0, obuf0)
    out_write(_UNITS - 2, obuf0, sem_o0)
    out_wait(obuf1, sem_o1)
    idx_wait(idx1, sem_i1)
    compute(idx1, obuf1)
    out_write(_UNITS - 1, obuf1, sem_o1)
    out_wait(obuf0, sem_o0)
    out_wait(obuf1, sem_o1)


@jax.jit
def kernel(sentence, table):
    sent_t = sentence.T
    table_t = table.T
    mesh = plsc.VectorSubcoreMesh(core_axis_name="c", subcore_axis_name="s")
    tmp = pl.kernel(
        _emb_kernel,
        out_type=jax.ShapeDtypeStruct((SEQ, EMB_DIM, BATCH), jnp.float32),
        mesh=mesh,
        scratch_types=[
            pltpu.VMEM((EMB_DIM, N_CHARS), jnp.float32),
            pltpu.VMEM((EMB_DIM * _FSTRIDE,), jnp.float32),
            pltpu.VMEM((_SROWS, 128), jnp.int32),
            pltpu.VMEM((_SROWS, 128), jnp.int32),
            pltpu.VMEM((_SROWS, EMB_DIM, 128), jnp.float32),
            pltpu.VMEM((_SROWS, EMB_DIM, 128), jnp.float32),
            pltpu.SemaphoreType.DMA,
            pltpu.SemaphoreType.DMA,
            pltpu.SemaphoreType.DMA,
            pltpu.SemaphoreType.DMA,
        ],
        compiler_params=pltpu.CompilerParams(
            use_tc_tiling_on_sc=True, needs_layout_passes=False),
    )(sent_t, table_t)
    return jnp.transpose(tmp, (2, 0, 1))
